# Initial kernel scaffold; baseline (speedup 1.0000x reference)
import jax
import jax.numpy as jnp
from jax.experimental import pallas as pl


def kernel(features, edge_index, W0, b0, al0, alb0, ar0, arb0, W1, b1, al1, alb1, ar1, arb1, Wf, bf, alf, albf, arf, arbf):
    raise NotImplementedError("write your pallas kernel here")


# speedup vs baseline: 4.5063x; 4.5063x over previous
import functools

import jax
import jax.numpy as jnp
import numpy as np
from jax.experimental import pallas as pl

N = 50000
E = 800000
HID = 64
HEADS = 4
NCLS = 32

_BN = 1000


def _prepare_body(x_ref, w_ref, b_ref, alr_ref, ab_ref, ft_ref, a_ref):
    x = x_ref[...]
    ft = jnp.dot(x, w_ref[...], preferred_element_type=jnp.float32) + b_ref[...]
    ft_ref[...] = ft
    a_ref[...] = jnp.dot(ft, alr_ref[...], preferred_element_type=jnp.float32) + ab_ref[...]


@functools.partial(jax.jit, static_argnames=("c_out",))
def _prepare(x, Wc, bc, ALR, ab, c_out):
    din = x.shape[1]
    grid = (N // _BN,)
    ft, a = pl.pallas_call(
        _prepare_body,
        grid=grid,
        in_specs=[
            pl.BlockSpec((_BN, din), lambda i: (i, 0)),
            pl.BlockSpec((din, c_out), lambda i: (0, 0)),
            pl.BlockSpec((1, c_out), lambda i: (0, 0)),
            pl.BlockSpec((c_out, 128), lambda i: (0, 0)),
            pl.BlockSpec((1, 128), lambda i: (0, 0)),
        ],
        out_specs=[
            pl.BlockSpec((_BN, c_out), lambda i: (i, 0)),
            pl.BlockSpec((_BN, 128), lambda i: (i, 0)),
        ],
        out_shape=[
            jax.ShapeDtypeStruct((N, c_out), jnp.float32),
            jax.ShapeDtypeStruct((N, 128), jnp.float32),
        ],
    )(x, Wc, bc, ALR, ab)
    return ft, a


def _edge_phase(ft, a1, a2, src, dst, nheads, hdim):
    e = a1[dst] + a2[src]
    e = jnp.where(e >= 0, e, 0.01 * e)
    m = jax.ops.segment_max(e, dst, num_segments=N)
    m = jnp.where(jnp.isfinite(m), m, 0.0)
    p = jnp.exp(e - m[dst])
    denom = jax.ops.segment_sum(p, dst, num_segments=N)
    ftg = ft[src].reshape(E, nheads, hdim)
    acc = jax.ops.segment_sum(
        (p[:, :, None] * ftg).reshape(E, nheads * hdim), dst, num_segments=N)
    acc = acc.reshape(N, nheads, hdim) / jnp.maximum(denom, 1e-16)[:, :, None]
    return acc.reshape(N, nheads * hdim)


def kernel(features, edge_index, W0, b0, al0, alb0, ar0, arb0, W1, b1, al1,
           alb1, ar1, arb1, Wf, bf, alf, albf, arf, arbf):
    src = edge_index[0]
    dst = edge_index[1]

    def layer_params(W, b, al, alb, ar, arb):
        H = W.shape[0]
        Wc = jnp.transpose(W, (1, 0, 2)).reshape(W.shape[1], H * HID)
        bc = b.reshape(1, H * HID)
        ALR = jnp.zeros((H * HID, 128), jnp.float32)
        for h in range(H):
            ALR = ALR.at[h * HID:(h + 1) * HID, h].set(al[h])
            ALR = ALR.at[h * HID:(h + 1) * HID, HEADS + h].set(ar[h])
        ab = jnp.zeros((1, 128), jnp.float32)
        ab = ab.at[0, :HEADS].set(alb)
        ab = ab.at[0, HEADS:2 * HEADS].set(arb)
        return Wc, bc, ALR, ab

    last = features
    for (W, b, al, alb, ar, arb) in ((W0, b0, al0, alb0, ar0, arb0),
                                     (W1, b1, al1, alb1, ar1, arb1)):
        Wc, bc, ALR, ab = layer_params(W, b, al, alb, ar, arb)
        ft, a = _prepare(last, Wc, bc, ALR, ab, HEADS * HID)
        a1 = a[:, :HEADS]
        a2 = a[:, HEADS:2 * HEADS]
        acc = _edge_phase(ft, a1, a2, src, dst, HEADS, HID)
        last = jax.nn.elu(acc)

    Wfp = jnp.zeros((HEADS * HID, 128), jnp.float32).at[:, :NCLS].set(Wf)
    bfp = jnp.zeros((1, 128), jnp.float32).at[0, :NCLS].set(bf)
    ALRf = jnp.zeros((128, 128), jnp.float32)
    ALRf = ALRf.at[:NCLS, 0].set(alf)
    ALRf = ALRf.at[:NCLS, 1].set(arf)
    abf = jnp.zeros((1, 128), jnp.float32)
    abf = abf.at[0, 0].set(albf)
    abf = abf.at[0, 1].set(arbf)
    ftf, af = _prepare(last, Wfp, bfp, ALRf, abf, 128)
    ftf = ftf[:, :NCLS]
    a1f = af[:, :1]
    a2f = af[:, 1:2]
    acc = _edge_phase(ftf, a1f, a2f, src, dst, 1, NCLS)
    return jax.nn.elu(acc)
